# baseline (speedup 1.0000x reference)
import math

import numpy as np
import jax
import jax.numpy as jnp
from jax.experimental import pallas as pl
from jax.experimental.pallas import tpu as pltpu

_H = 6
_WS = 8
_N = _WS * _WS
_P = _N * _N
_T = (2 * _WS - 1) ** 2
_RB = 2048
_LS = 128


def _build_tables():
    ch = np.arange(-(_WS - 1), _WS, dtype=np.float32)
    t = np.stack(np.meshgrid(ch, ch, indexing="ij"), axis=-1)
    t /= float(_WS - 1)
    t *= 8.0
    t = np.sign(t) * np.log2(np.abs(t) + 1.0) / np.log2(8.0)
    coords = t.reshape(_T, 2)

    c = np.arange(_WS)
    grid = np.stack(np.meshgrid(c, c, indexing="ij")).reshape(2, -1)
    rel = (grid[:, :, None] - grid[:, None, :]).transpose(1, 2, 0)
    rel = rel.astype(np.int64)
    rel[:, :, 0] += _WS - 1
    rel[:, :, 1] += _WS - 1
    rel[:, :, 0] *= 2 * _WS - 1
    idx = rel.sum(-1).reshape(-1)

    onehot = np.zeros((_P, _T), dtype=np.float32)
    onehot[np.arange(_P), idx] = 1.0
    return coords, onehot


_TC_NP, _OC_NP = _build_tables()


def _fused_kernel(ls_ref, w1_ref, b1_ref, w2_ref, tc_ref, oc_ref, attn_ref,
                  out_ref, bvt_vmem, scale_vmem):
    hd = pl.program_id(0)
    i = pl.program_id(1)

    @pl.when(jnp.logical_and(hd == 0, i == 0))
    def _prologue():
        h = jnp.dot(tc_ref[...], w1_ref[...],
                    preferred_element_type=jnp.float32)
        h = jnp.maximum(h + b1_ref[...], 0.0)
        tbl = jax.lax.dot_general(h, w2_ref[...],
                                  (((1,), (1,)), ((), ())),
                                  preferred_element_type=jnp.float32)
        tbl = 16.0 * jax.nn.sigmoid(tbl)
        tbl_hi = tbl.astype(jnp.bfloat16)
        tbl_lo = (tbl - tbl_hi.astype(jnp.float32)).astype(jnp.bfloat16)
        oc = oc_ref[...]
        bvt_vmem[...] = (
            jnp.dot(oc, tbl_hi, preferred_element_type=jnp.float32) +
            jnp.dot(oc, tbl_lo, preferred_element_type=jnp.float32))
        scale_vmem[...] = jnp.exp(jnp.minimum(ls_ref[...], math.log(100.0)))

    oh = (jax.lax.broadcasted_iota(jnp.int32, (1, _H), 1) == hd
          ).astype(jnp.float32)
    bcols = bvt_vmem[pl.ds(i * _RB, _RB), :]
    bh = jnp.broadcast_to(
        jnp.sum(bcols * oh, axis=1, keepdims=True), (_RB, _LS))
    sh = jnp.sum(scale_vmem[...] * oh, axis=1, keepdims=True)
    nlt = attn_ref.shape[2] // _LS
    for lt in range(nlt):
        sl = slice(lt * _LS, (lt + 1) * _LS)
        out_ref[0, :, sl] = attn_ref[0, :, sl] * sh + bh


def kernel(attn, x_size, logit_scale, w1, b1, w2):
    del x_size
    B = attn.shape[0]
    attn_t = jnp.transpose(attn, (1, 2, 3, 0)).reshape(_H, _P, B)

    tc = jnp.asarray(_TC_NP)
    oc = jnp.asarray(_OC_NP, dtype=jnp.bfloat16)
    ls2 = logit_scale.reshape(1, _H)
    b1r = b1.reshape(1, -1)

    out_t = pl.pallas_call(
        _fused_kernel,
        grid=(_H, _P // _RB),
        in_specs=[
            pl.BlockSpec((1, _H), lambda h, i: (0, 0)),
            pl.BlockSpec((2, 512), lambda h, i: (0, 0)),
            pl.BlockSpec((1, 512), lambda h, i: (0, 0)),
            pl.BlockSpec((_H, 512), lambda h, i: (0, 0)),
            pl.BlockSpec((_T, 2), lambda h, i: (0, 0)),
            pl.BlockSpec((_P, _T), lambda h, i: (0, 0)),
            pl.BlockSpec((1, _RB, B), lambda h, i: (h, i, 0)),
        ],
        out_specs=pl.BlockSpec((1, _RB, B), lambda h, i: (h, i, 0)),
        out_shape=jax.ShapeDtypeStruct((_H, _P, B), jnp.float32),
        scratch_shapes=[
            pltpu.VMEM((_P, _H), jnp.float32),
            pltpu.VMEM((1, _H), jnp.float32),
        ],
        compiler_params=pltpu.CompilerParams(
            dimension_semantics=("arbitrary", "arbitrary"),
            vmem_limit_bytes=60 * 1024 * 1024,
        ),
    )(ls2, w1, b1r, w2.T, tc, oc, attn_t)
    return jnp.transpose(out_t.reshape(_H, _N, _N, B), (3, 0, 1, 2))


# speedup vs baseline: 1.0251x; 1.0251x over previous
import math

import numpy as np
import jax
import jax.numpy as jnp
from jax.experimental import pallas as pl
from jax.experimental.pallas import tpu as pltpu

_H = 6
_WS = 8
_N = _WS * _WS
_P = _N * _N
_T = (2 * _WS - 1) ** 2
_RB = 256
_LS = 128


def _build_tables():
    ch = np.arange(-(_WS - 1), _WS, dtype=np.float32)
    t = np.stack(np.meshgrid(ch, ch, indexing="ij"), axis=-1)
    t /= float(_WS - 1)
    t *= 8.0
    t = np.sign(t) * np.log2(np.abs(t) + 1.0) / np.log2(8.0)
    coords = t.reshape(_T, 2)

    c = np.arange(_WS)
    grid = np.stack(np.meshgrid(c, c, indexing="ij")).reshape(2, -1)
    rel = (grid[:, :, None] - grid[:, None, :]).transpose(1, 2, 0)
    rel = rel.astype(np.int64)
    rel[:, :, 0] += _WS - 1
    rel[:, :, 1] += _WS - 1
    rel[:, :, 0] *= 2 * _WS - 1
    idx = rel.sum(-1).reshape(-1)

    onehot = np.zeros((_P, _T), dtype=np.float32)
    onehot[np.arange(_P), idx] = 1.0
    return coords, onehot


_TC_NP, _OC_NP = _build_tables()


def _fused_kernel(ls_ref, w1_ref, b1_ref, w2_ref, tc_ref, oc_ref, attn_ref,
                  out_ref, bvt_vmem, scale_vmem):
    i = pl.program_id(0)

    @pl.when(i == 0)
    def _prologue():
        h = jnp.dot(tc_ref[...], w1_ref[...],
                    preferred_element_type=jnp.float32)
        h = jnp.maximum(h + b1_ref[...], 0.0)
        tbl = jax.lax.dot_general(h, w2_ref[...],
                                  (((1,), (1,)), ((), ())),
                                  preferred_element_type=jnp.float32)
        tbl = 16.0 * jax.nn.sigmoid(tbl)
        tbl_hi = tbl.astype(jnp.bfloat16)
        tbl_lo = (tbl - tbl_hi.astype(jnp.float32)).astype(jnp.bfloat16)
        oc = oc_ref[...]
        bvt_vmem[...] = (
            jnp.dot(oc, tbl_hi, preferred_element_type=jnp.float32) +
            jnp.dot(oc, tbl_lo, preferred_element_type=jnp.float32))
        sc = jnp.exp(jnp.minimum(ls_ref[...], math.log(100.0)))
        scale_vmem[...] = jnp.transpose(sc, (1, 0))

    nlt = attn_ref.shape[2] // _LS
    for hd in range(_H):
        bh = jnp.broadcast_to(
            bvt_vmem[pl.ds(i * _RB, _RB), hd:hd + 1], (_RB, _LS))
        sh = scale_vmem[hd, 0]
        for lt in range(nlt):
            sl = slice(lt * _LS, (lt + 1) * _LS)
            out_ref[hd, :, sl] = attn_ref[hd, :, sl] * sh + bh


def kernel(attn, x_size, logit_scale, w1, b1, w2):
    del x_size
    B = attn.shape[0]
    attn_t = jnp.transpose(attn, (1, 2, 3, 0)).reshape(_H, _P, B)

    tc = jnp.asarray(_TC_NP)
    oc = jnp.asarray(_OC_NP, dtype=jnp.bfloat16)
    ls2 = logit_scale.reshape(1, _H)
    b1r = b1.reshape(1, -1)

    out_t = pl.pallas_call(
        _fused_kernel,
        grid=(_P // _RB,),
        in_specs=[
            pl.BlockSpec((1, _H), lambda i: (0, 0)),
            pl.BlockSpec((2, 512), lambda i: (0, 0)),
            pl.BlockSpec((1, 512), lambda i: (0, 0)),
            pl.BlockSpec((_H, 512), lambda i: (0, 0)),
            pl.BlockSpec((_T, 2), lambda i: (0, 0)),
            pl.BlockSpec((_P, _T), lambda i: (0, 0)),
            pl.BlockSpec((_H, _RB, B), lambda i: (0, i, 0)),
        ],
        out_specs=pl.BlockSpec((_H, _RB, B), lambda i: (0, i, 0)),
        out_shape=jax.ShapeDtypeStruct((_H, _P, B), jnp.float32),
        scratch_shapes=[
            pltpu.VMEM((_P, _H), jnp.float32),
            pltpu.VMEM((_H, 1), jnp.float32),
        ],
        compiler_params=pltpu.CompilerParams(
            dimension_semantics=("arbitrary",),
            vmem_limit_bytes=60 * 1024 * 1024,
        ),
    )(ls2, w1, b1r, w2.T, tc, oc, attn_t)
    return jnp.transpose(out_t.reshape(_H, _N, _N, B), (3, 0, 1, 2))


# speedup vs baseline: 1.0683x; 1.0421x over previous
import math

import numpy as np
import jax
import jax.numpy as jnp
from jax.experimental import pallas as pl
from jax.experimental.pallas import tpu as pltpu

_H = 6
_WS = 8
_N = _WS * _WS
_P = _N * _N
_T = (2 * _WS - 1) ** 2
_RB = 512
_LS = 128


def _build_tables():
    ch = np.arange(-(_WS - 1), _WS, dtype=np.float32)
    t = np.stack(np.meshgrid(ch, ch, indexing="ij"), axis=-1)
    t /= float(_WS - 1)
    t *= 8.0
    t = np.sign(t) * np.log2(np.abs(t) + 1.0) / np.log2(8.0)
    coords = t.reshape(_T, 2)

    c = np.arange(_WS)
    grid = np.stack(np.meshgrid(c, c, indexing="ij")).reshape(2, -1)
    rel = (grid[:, :, None] - grid[:, None, :]).transpose(1, 2, 0)
    rel = rel.astype(np.int64)
    rel[:, :, 0] += _WS - 1
    rel[:, :, 1] += _WS - 1
    rel[:, :, 0] *= 2 * _WS - 1
    idx = rel.sum(-1).reshape(-1)

    onehot = np.zeros((_P, _T), dtype=np.float32)
    onehot[np.arange(_P), idx] = 1.0
    return coords, onehot


_TC_NP, _OC_NP = _build_tables()


def _fused_kernel(ls_ref, w1_ref, b1_ref, w2_ref, tc_ref, oc_ref, attn_ref,
                  out_ref, bvt_vmem, scale_vmem):
    i = pl.program_id(0)

    @pl.when(i == 0)
    def _prologue():
        h = jnp.dot(tc_ref[...], w1_ref[...],
                    preferred_element_type=jnp.float32)
        h = jnp.maximum(h + b1_ref[...], 0.0)
        tbl = jax.lax.dot_general(h, w2_ref[...],
                                  (((1,), (1,)), ((), ())),
                                  preferred_element_type=jnp.float32)
        tbl = 16.0 * jax.nn.sigmoid(tbl)
        tbl_hi = tbl.astype(jnp.bfloat16)
        tbl_lo = (tbl - tbl_hi.astype(jnp.float32)).astype(jnp.bfloat16)
        oc = oc_ref[...]
        bvt_vmem[...] = (
            jnp.dot(oc, tbl_hi, preferred_element_type=jnp.float32) +
            jnp.dot(oc, tbl_lo, preferred_element_type=jnp.float32))
        sc = jnp.exp(jnp.minimum(ls_ref[...], math.log(100.0)))
        scale_vmem[...] = jnp.transpose(sc, (1, 0))

    nb = attn_ref.shape[2]
    for hd in range(_H):
        bh = jnp.broadcast_to(
            bvt_vmem[pl.ds(i * _RB, _RB), hd:hd + 1], (_RB, nb))
        sh = scale_vmem[hd, 0]
        out_ref[hd] = attn_ref[hd] * sh + bh


def kernel(attn, x_size, logit_scale, w1, b1, w2):
    del x_size
    B = attn.shape[0]
    attn_t = jnp.transpose(attn, (1, 2, 3, 0)).reshape(_H, _P, B)

    tc = jnp.asarray(_TC_NP)
    oc = jnp.asarray(_OC_NP, dtype=jnp.bfloat16)
    ls2 = logit_scale.reshape(1, _H)
    b1r = b1.reshape(1, -1)

    out_t = pl.pallas_call(
        _fused_kernel,
        grid=(_P // _RB,),
        in_specs=[
            pl.BlockSpec((1, _H), lambda i: (0, 0)),
            pl.BlockSpec((2, 512), lambda i: (0, 0)),
            pl.BlockSpec((1, 512), lambda i: (0, 0)),
            pl.BlockSpec((_H, 512), lambda i: (0, 0)),
            pl.BlockSpec((_T, 2), lambda i: (0, 0)),
            pl.BlockSpec((_P, _T), lambda i: (0, 0)),
            pl.BlockSpec((_H, _RB, B), lambda i: (0, i, 0)),
        ],
        out_specs=pl.BlockSpec((_H, _RB, B), lambda i: (0, i, 0)),
        out_shape=jax.ShapeDtypeStruct((_H, _P, B), jnp.float32),
        scratch_shapes=[
            pltpu.VMEM((_P, _H), jnp.float32),
            pltpu.VMEM((_H, 1), jnp.float32),
        ],
        compiler_params=pltpu.CompilerParams(
            dimension_semantics=("arbitrary",),
            vmem_limit_bytes=60 * 1024 * 1024,
        ),
    )(ls2, w1, b1r, w2.T, tc, oc, attn_t)
    return jnp.transpose(out_t.reshape(_H, _N, _N, B), (3, 0, 1, 2))
